# baseline (speedup 1.0000x reference)
import functools

import jax
import jax.numpy as jnp
from jax import lax
from jax.experimental import pallas as pl
from jax.experimental.pallas import tpu as pltpu
from jax.experimental.pallas import tpu_sc as plsc

NC = 2
NSB = 5
NS = 16
K = 80
LANES = 16


def _proj_body(x_ref, w_ref, b_ref, y_ref):
    y_ref[...] = (
        lax.dot_general(x_ref[...], w_ref[0], (((1,), (1,)), ((), ())),
                        preferred_element_type=jnp.float32)
        + b_ref[0]
    )


def _project_all(x, rel_W, rel_b, by=1000):
    n, d = x.shape
    r = rel_W.shape[0]
    nby = n // by
    return pl.pallas_call(
        _proj_body,
        grid=(nby, r),
        in_specs=[
            pl.BlockSpec((by, d), lambda i, rr: (i, 0)),
            pl.BlockSpec((1, d, d), lambda i, rr: (rr, 0, 0)),
            pl.BlockSpec((1, 1, d), lambda i, rr: (rr, 0, 0)),
        ],
        out_specs=pl.BlockSpec((by, d), lambda i, rr: (rr * nby + i, 0)),
        out_shape=jax.ShapeDtypeStruct((r * n, d), jnp.float32),
    )(x, rel_W, rel_b.reshape(r, 1, d))


def _gidx_body(ei_ref, t_ref, g_ref, d_ref):
    n = _gidx_body.n
    g_ref[...] = t_ref[...] * n + ei_ref[0]
    d_ref[...] = ei_ref[1]


def _gidx(ei5, typ4, n):
    _, nw, nsb, sbc, k = ei5.shape
    _gidx_body.n = n
    shp = jax.ShapeDtypeStruct((nw, nsb, sbc, k), jnp.int32)
    return pl.pallas_call(
        _gidx_body,
        grid=(nw // 8,),
        in_specs=[pl.BlockSpec((2, 8, nsb, sbc, k), lambda i: (0, i, 0, 0, 0)),
                  pl.BlockSpec((8, nsb, sbc, k), lambda i: (i, 0, 0, 0))],
        out_specs=[pl.BlockSpec((8, nsb, sbc, k), lambda i: (i, 0, 0, 0)),
                   pl.BlockSpec((8, nsb, sbc, k), lambda i: (i, 0, 0, 0))],
        out_shape=[shp, shp],
    )(ei5, typ4)


def _sc_aggregate(y, gidx2, dst2, n, d):
    sb = gidx2.shape[2]
    zr = 1000
    nzt = n // zr
    mesh = plsc.VectorSubcoreMesh(core_axis_name="c", subcore_axis_name="s",
                                  num_cores=NC)

    @functools.partial(
        pl.kernel,
        out_type=[
            jax.ShapeDtypeStruct((NC, n, d), jnp.float32),
            jax.ShapeDtypeStruct((NC, 1, n), jnp.float32),
        ],
        mesh=mesh,
    scratch_types=[
            pltpu.VMEM((sb, K), jnp.int32),
            pltpu.VMEM((sb, K), jnp.int32),
            pltpu.VMEM((K, d), jnp.float32),
            pltpu.VMEM((K, d), jnp.float32),
            pltpu.VMEM((K, d), jnp.float32),
            pltpu.VMEM((K, d), jnp.float32),
            pltpu.VMEM((K,), jnp.float32),
            pltpu.VMEM_SHARED((n, d), jnp.float32),
            pltpu.VMEM_SHARED((n,), jnp.float32),
            pltpu.SemaphoreType.DMA,
            pltpu.SemaphoreType.DMA,
            pltpu.SemaphoreType.DMA,
            pltpu.SemaphoreType.DMA,
            pltpu.SemaphoreType.DMA,
            pltpu.SemaphoreType.DMA,
            pltpu.SemaphoreType.DMA,
            pltpu.SemaphoreType.DMA,
            pltpu.SemaphoreType.DMA,
        ],
    )
    def sc_kernel(y_hbm, gidx_hbm, dst_hbm, z2_hbm, z1_hbm,
                  acc_out, deg_out,
                  gidx_v, dst_v, rows0_v, rows1_v, rows2_v, rows3_v, ones_v,
                  acc_sh, deg_sh, semg0, semg1, semg2, semg3,
                  sems0, sems1, sems2, sems3, semd):
        c = lax.axis_index("c")
        s = lax.axis_index("s")
        widx = c * NS + s

        arow = pl.multiple_of(s * zr, 8)

        @pl.when(s < nzt)
        def _():
            pltpu.sync_copy(z2_hbm, acc_sh.at[pl.ds(arow, zr)])

        @pl.when(s == 0)
        def _():
            pltpu.sync_copy(z1_hbm, deg_sh)

        for k in range(K // LANES):
            ones_v[pl.ds(k * LANES, LANES)] = jnp.ones((LANES,), jnp.float32)

        plsc.subcore_barrier()

        bufs = (rows0_v, rows1_v, rows2_v, rows3_v)
        semg = (semg0, semg1, semg2, semg3)
        sems = (sems0, sems1, sems2, sems3)

        def wait_g(j, p):
            pltpu.make_async_copy(y_hbm.at[gidx_v.at[j]], bufs[p],
                                  semg[p]).wait()

        def issue_s(j, p):
            pltpu.async_copy(bufs[p], acc_sh.at[dst_v.at[j]], sems[p],
                             add=True)
            pltpu.async_copy(ones_v, deg_sh.at[dst_v.at[j]], semd, add=True)

        def wait_s(j, p):
            pltpu.make_async_copy(bufs[p], acc_sh.at[dst_v.at[j]],
                                  sems[p]).wait()

        def outer(b, carry):
            pltpu.sync_copy(gidx_hbm.at[widx, b], gidx_v)
            pltpu.sync_copy(dst_hbm.at[widx, b], dst_v)
            pltpu.async_copy(y_hbm.at[gidx_v.at[0]], rows0_v, semg0)
            pltpu.async_copy(y_hbm.at[gidx_v.at[1]], rows1_v, semg1)
            pltpu.async_copy(y_hbm.at[gidx_v.at[2]], rows2_v, semg2)
            pltpu.async_copy(y_hbm.at[gidx_v.at[3]], rows3_v, semg3)
            wait_g(0, 0)
            issue_s(0, 0)

            def quad(i, carry2):
                for t in range(4):
                    j = i * 4 + 1 + t
                    p = (1 + t) % 4
                    q = t % 4
                    wait_s(j - 1, q)

                    @pl.when(j + 3 < sb)
                    def _():
                        pltpu.async_copy(y_hbm.at[gidx_v.at[j + 3]],
                                         bufs[q], semg[q])
                    wait_g(j, p)
                    issue_s(j, p)
                return carry2
            lax.fori_loop(0, (sb - 1) // 4, quad, 0, unroll=False)
            wait_s(sb - 1, (sb - 1) % 4)

            def drain(j, carry2):
                pltpu.make_async_copy(ones_v, deg_sh.at[dst_v.at[j]],
                                      semd).wait()
                return carry2
            lax.fori_loop(0, sb, drain, 0, unroll=False)
            return carry
        lax.fori_loop(0, NSB, outer, 0, unroll=False)

        plsc.subcore_barrier()

        @pl.when(s < nzt)
        def _():
            pltpu.sync_copy(acc_sh.at[pl.ds(arow, zr)],
                            acc_out.at[c, pl.ds(arow, zr)])

        @pl.when(s == 0)
        def _():
            pltpu.sync_copy(deg_sh, deg_out.at[c, 0])

    z2 = jnp.zeros((zr, d), jnp.float32)
    z1 = jnp.zeros((n,), jnp.float32)
    return sc_kernel(y, gidx2, dst2, z2, z1)


def _ln(t, g, b, eps=1e-5):
    m = jnp.mean(t, axis=-1, keepdims=True)
    v = jnp.mean((t - m) ** 2, axis=-1, keepdims=True)
    return (t - m) / jnp.sqrt(v + eps) * g + b


def _tail_body(x_ref, a_ref, d_ref, sw_ref, sb_ref, g1_ref, bb1_ref,
               w1_ref, b1_ref, w2_ref, b2_ref, g2_ref, bb2_ref, o_ref):
    xb = x_ref[...]
    deg = jnp.maximum(sum(d_ref[i] for i in range(NC)), 1.0)
    agg = sum(a_ref[i] for i in range(NC)) / deg
    h = (lax.dot_general(xb, sw_ref[...], (((1,), (1,)), ((), ())),
                         preferred_element_type=jnp.float32)
         + sb_ref[...] + agg)
    x1 = _ln(xb + h, g1_ref[...], bb1_ref[...])
    u = (lax.dot_general(x1, w1_ref[...], (((1,), (1,)), ((), ())),
                         preferred_element_type=jnp.float32)
         + b1_ref[...])
    gl = 0.5 * u * (1.0 + lax.erf(u * 0.7071067811865476))
    ff = (lax.dot_general(gl, w2_ref[...], (((1,), (1,)), ((), ())),
                          preferred_element_type=jnp.float32)
          + b2_ref[...])
    o_ref[...] = _ln(x1 + ff, g2_ref[...], bb2_ref[...])


def _tail(x, accp, degp, self_W, self_b, ln1_g, ln1_b, W1, b1, W2, b2,
          ln2_g, ln2_b, bn=1000):
    n, d = x.shape
    h2 = W1.shape[0]
    nb = n // bn
    full = lambda *shape: pl.BlockSpec(shape, lambda i: (0,) * len(shape))
    return pl.pallas_call(
        _tail_body,
        grid=(nb,),
        in_specs=[
            pl.BlockSpec((bn, d), lambda i: (i, 0)),
            pl.BlockSpec((NC, bn, d), lambda i: (0, i, 0)),
            pl.BlockSpec((NC, bn, 1), lambda i: (0, i, 0)),
            full(d, d), full(1, d), full(1, d), full(1, d),
            full(h2, d), full(1, h2), full(d, h2), full(1, d),
            full(1, d), full(1, d),
        ],
        out_specs=pl.BlockSpec((bn, d), lambda i: (i, 0)),
        out_shape=jax.ShapeDtypeStruct((n, d), jnp.float32),
    )(x, accp, degp, self_W, self_b.reshape(1, d), ln1_g.reshape(1, d),
      ln1_b.reshape(1, d), W1, b1.reshape(1, h2), W2, b2.reshape(1, d),
      ln2_g.reshape(1, d), ln2_b.reshape(1, d))


def kernel(x, edge_index, edge_type, self_W, self_b, rel_W, rel_b,
           ln1_g, ln1_b, W1, b1, W2, b2, ln2_g, ln2_b):
    n, d = x.shape
    e = edge_type.shape[0]
    y = _project_all(x, rel_W, rel_b)
    nw = NC * NS
    gidx2, dst2 = _gidx(edge_index.reshape(2, nw, NSB, -1, K),
                        edge_type.reshape(nw, NSB, -1, K), n)
    accp, degp = _sc_aggregate(y, gidx2, dst2, n, d)
    return _tail(x, accp, degp.reshape(NC, n, 1), self_W, self_b,
                 ln1_g, ln1_b, W1, b1, W2, b2, ln2_g, ln2_b)


# speedup vs baseline: 1.0283x; 1.0283x over previous
import functools

import jax
import jax.numpy as jnp
from jax import lax
from jax.experimental import pallas as pl
from jax.experimental.pallas import tpu as pltpu
from jax.experimental.pallas import tpu_sc as plsc

NC = 2
NSB = 5
NS = 16
K = 80
LANES = 16


def _proj_body(x_ref, w_ref, b_ref, y_ref):
    y_ref[...] = (
        lax.dot_general(x_ref[...], w_ref[0], (((1,), (1,)), ((), ())),
                        preferred_element_type=jnp.float32)
        + b_ref[0]
    )


def _project_all(x, rel_W, rel_b, by=1000):
    n, d = x.shape
    r = rel_W.shape[0]
    nby = n // by
    return pl.pallas_call(
        _proj_body,
        grid=(nby, r),
        in_specs=[
            pl.BlockSpec((by, d), lambda i, rr: (i, 0)),
            pl.BlockSpec((1, d, d), lambda i, rr: (rr, 0, 0)),
            pl.BlockSpec((1, 1, d), lambda i, rr: (rr, 0, 0)),
        ],
        out_specs=pl.BlockSpec((by, d), lambda i, rr: (rr * nby + i, 0)),
        out_shape=jax.ShapeDtypeStruct((r * n, d), jnp.float32),
    )(x, rel_W, rel_b.reshape(r, 1, d))


def _gidx_body(ei_ref, t_ref, g_ref, d_ref):
    n = _gidx_body.n
    g_ref[...] = t_ref[...] * n + ei_ref[0]
    d_ref[...] = ei_ref[1]


def _gidx(ei5, typ4, n):
    _, nw, nsb, sbc, k = ei5.shape
    _gidx_body.n = n
    shp = jax.ShapeDtypeStruct((nw, nsb, sbc, k), jnp.int32)
    return pl.pallas_call(
        _gidx_body,
        grid=(nw // 8,),
        in_specs=[pl.BlockSpec((2, 8, nsb, sbc, k), lambda i: (0, i, 0, 0, 0)),
                  pl.BlockSpec((8, nsb, sbc, k), lambda i: (i, 0, 0, 0))],
        out_specs=[pl.BlockSpec((8, nsb, sbc, k), lambda i: (i, 0, 0, 0)),
                   pl.BlockSpec((8, nsb, sbc, k), lambda i: (i, 0, 0, 0))],
        out_shape=[shp, shp],
    )(ei5, typ4)


def _sc_aggregate(y, gidx2, dst2, n, d):
    sb = gidx2.shape[2]
    zr = 1000
    nzt = n // zr
    mesh = plsc.VectorSubcoreMesh(core_axis_name="c", subcore_axis_name="s",
                                  num_cores=NC)

    @functools.partial(
        pl.kernel,
        out_type=[
            jax.ShapeDtypeStruct((NC, n, d), jnp.float32),
            jax.ShapeDtypeStruct((NC, 1, n), jnp.float32),
        ],
        mesh=mesh,
    scratch_types=[
            pltpu.VMEM((sb, K), jnp.int32),
            pltpu.VMEM((sb, K), jnp.int32),
            pltpu.VMEM((sb, K), jnp.int32),
            pltpu.VMEM((sb, K), jnp.int32),
            pltpu.VMEM((K, d), jnp.float32),
            pltpu.VMEM((K, d), jnp.float32),
            pltpu.VMEM((K, d), jnp.float32),
            pltpu.VMEM((K,), jnp.float32),
            pltpu.VMEM_SHARED((n, d), jnp.float32),
            pltpu.VMEM_SHARED((n,), jnp.float32),
            pltpu.SemaphoreType.DMA,
            pltpu.SemaphoreType.DMA,
            pltpu.SemaphoreType.DMA,
            pltpu.SemaphoreType.DMA,
            pltpu.SemaphoreType.DMA,
            pltpu.SemaphoreType.DMA,
            pltpu.SemaphoreType.DMA,
            pltpu.SemaphoreType.DMA,
        ],
    )
    def sc_kernel(y_hbm, gidx_hbm, dst_hbm, z2_hbm, z1_hbm,
                  acc_out, deg_out,
                  gidx0_v, gidx1_v, dst0_v, dst1_v,
                  rows0_v, rows1_v, rows2_v, ones_v,
                  acc_sh, deg_sh, semg0, semg1, semg2,
                  sems0, sems1, sems2, semd, semi):
        c = lax.axis_index("c")
        s = lax.axis_index("s")
        widx = c * NS + s

        arow = pl.multiple_of(s * zr, 8)

        @pl.when(s < nzt)
        def _():
            pltpu.sync_copy(z2_hbm, acc_sh.at[pl.ds(arow, zr)])

        @pl.when(s == 0)
        def _():
            pltpu.sync_copy(z1_hbm, deg_sh)

        for k in range(K // LANES):
            ones_v[pl.ds(k * LANES, LANES)] = jnp.ones((LANES,), jnp.float32)

        plsc.subcore_barrier()

        bufs = (rows0_v, rows1_v, rows2_v)
        semg = (semg0, semg1, semg2)
        sems = (sems0, sems1, sems2)
        gstage = (gidx0_v, gidx1_v)
        dstage = (dst0_v, dst1_v)

        def issue_stage(b, w):
            pltpu.async_copy(gidx_hbm.at[widx, b], gstage[w], semi)
            pltpu.async_copy(dst_hbm.at[widx, b], dstage[w], semi)

        def wait_stage(b, w):
            pltpu.make_async_copy(gidx_hbm.at[widx, b], gstage[w], semi).wait()
            pltpu.make_async_copy(dst_hbm.at[widx, b], dstage[w], semi).wait()

        issue_stage(0, 0)
        wait_stage(0, 0)

        for b in range(NSB):
            w = b % 2
            gidx_v = gstage[w]
            dst_v = dstage[w]
            if b + 1 < NSB:
                issue_stage(b + 1, 1 - w)

            def wait_g(j, p):
                pltpu.make_async_copy(y_hbm.at[gidx_v.at[j]], bufs[p],
                                      semg[p]).wait()

            def issue_s(j, p):
                pltpu.async_copy(bufs[p], acc_sh.at[dst_v.at[j]], sems[p],
                                 add=True)
                pltpu.async_copy(ones_v, deg_sh.at[dst_v.at[j]], semd,
                                 add=True)

            def wait_s(j, p):
                pltpu.make_async_copy(bufs[p], acc_sh.at[dst_v.at[j]],
                                      sems[p]).wait()

            pltpu.async_copy(y_hbm.at[gidx_v.at[0]], rows0_v, semg0)
            pltpu.async_copy(y_hbm.at[gidx_v.at[1]], rows1_v, semg1)
            pltpu.async_copy(y_hbm.at[gidx_v.at[2]], rows2_v, semg2)
            wait_g(0, 0)
            issue_s(0, 0)

            def tri(i, carry2):
                for t in range(3):
                    j = i * 3 + 1 + t
                    p = (1 + t) % 3
                    q = t % 3
                    wait_s(j - 1, q)

                    @pl.when(j + 2 < sb)
                    def _():
                        pltpu.async_copy(y_hbm.at[gidx_v.at[j + 2]],
                                         bufs[q], semg[q])
                    wait_g(j, p)
                    issue_s(j, p)
                return carry2
            lax.fori_loop(0, (sb - 1) // 3, tri, 0, unroll=False)
            wait_s(sb - 1, (sb - 1) % 3)

            def drain(j, carry2):
                pltpu.make_async_copy(ones_v, deg_sh.at[dst_v.at[j]],
                                      semd).wait()
                return carry2
            lax.fori_loop(0, sb, drain, 0, unroll=False)
            if b + 1 < NSB:
                wait_stage(b + 1, 1 - w)

        plsc.subcore_barrier()

        @pl.when(s < nzt)
        def _():
            pltpu.sync_copy(acc_sh.at[pl.ds(arow, zr)],
                            acc_out.at[c, pl.ds(arow, zr)])

        @pl.when(s == 0)
        def _():
            pltpu.sync_copy(deg_sh, deg_out.at[c, 0])

    z2 = jnp.zeros((zr, d), jnp.float32)
    z1 = jnp.zeros((n,), jnp.float32)
    return sc_kernel(y, gidx2, dst2, z2, z1)


def _ln(t, g, b, eps=1e-5):
    m = jnp.mean(t, axis=-1, keepdims=True)
    v = jnp.mean((t - m) ** 2, axis=-1, keepdims=True)
    return (t - m) / jnp.sqrt(v + eps) * g + b


def _tail_body(x_ref, a_ref, d_ref, sw_ref, sb_ref, g1_ref, bb1_ref,
               w1_ref, b1_ref, w2_ref, b2_ref, g2_ref, bb2_ref, o_ref):
    xb = x_ref[...]
    deg = jnp.maximum(sum(d_ref[i] for i in range(NC)), 1.0)
    agg = sum(a_ref[i] for i in range(NC)) / deg
    h = (lax.dot_general(xb, sw_ref[...], (((1,), (1,)), ((), ())),
                         preferred_element_type=jnp.float32)
         + sb_ref[...] + agg)
    x1 = _ln(xb + h, g1_ref[...], bb1_ref[...])
    u = (lax.dot_general(x1, w1_ref[...], (((1,), (1,)), ((), ())),
                         preferred_element_type=jnp.float32)
         + b1_ref[...])
    gl = 0.5 * u * (1.0 + lax.erf(u * 0.7071067811865476))
    ff = (lax.dot_general(gl, w2_ref[...], (((1,), (1,)), ((), ())),
                          preferred_element_type=jnp.float32)
          + b2_ref[...])
    o_ref[...] = _ln(x1 + ff, g2_ref[...], bb2_ref[...])


def _tail(x, accp, degp, self_W, self_b, ln1_g, ln1_b, W1, b1, W2, b2,
          ln2_g, ln2_b, bn=1000):
    n, d = x.shape
    h2 = W1.shape[0]
    nb = n // bn
    full = lambda *shape: pl.BlockSpec(shape, lambda i: (0,) * len(shape))
    return pl.pallas_call(
        _tail_body,
        grid=(nb,),
        in_specs=[
            pl.BlockSpec((bn, d), lambda i: (i, 0)),
            pl.BlockSpec((NC, bn, d), lambda i: (0, i, 0)),
            pl.BlockSpec((NC, bn, 1), lambda i: (0, i, 0)),
            full(d, d), full(1, d), full(1, d), full(1, d),
            full(h2, d), full(1, h2), full(d, h2), full(1, d),
            full(1, d), full(1, d),
        ],
        out_specs=pl.BlockSpec((bn, d), lambda i: (i, 0)),
        out_shape=jax.ShapeDtypeStruct((n, d), jnp.float32),
    )(x, accp, degp, self_W, self_b.reshape(1, d), ln1_g.reshape(1, d),
      ln1_b.reshape(1, d), W1, b1.reshape(1, h2), W2, b2.reshape(1, d),
      ln2_g.reshape(1, d), ln2_b.reshape(1, d))


def kernel(x, edge_index, edge_type, self_W, self_b, rel_W, rel_b,
           ln1_g, ln1_b, W1, b1, W2, b2, ln2_g, ln2_b):
    n, d = x.shape
    e = edge_type.shape[0]
    y = _project_all(x, rel_W, rel_b)
    nw = NC * NS
    gidx2, dst2 = _gidx(edge_index.reshape(2, nw, NSB, -1, K),
                        edge_type.reshape(nw, NSB, -1, K), n)
    accp, degp = _sc_aggregate(y, gidx2, dst2, n, d)
    return _tail(x, accp, degp.reshape(NC, n, 1), self_W, self_b,
                 ln1_g, ln1_b, W1, b1, W2, b2, ln2_g, ln2_b)


# speedup vs baseline: 1.0328x; 1.0043x over previous
import functools

import jax
import jax.numpy as jnp
from jax import lax
from jax.experimental import pallas as pl
from jax.experimental.pallas import tpu as pltpu
from jax.experimental.pallas import tpu_sc as plsc

NC = 2
NSB = 5
NS = 16
K = 80
LANES = 16


def _proj_body(x_ref, w_ref, b_ref, y_ref):
    y_ref[...] = (
        lax.dot_general(x_ref[...], w_ref[0], (((1,), (1,)), ((), ())),
                        preferred_element_type=jnp.float32)
        + b_ref[0]
    )


def _project_all(x, rel_W, rel_b, by=1000):
    n, d = x.shape
    r = rel_W.shape[0]
    nby = n // by
    return pl.pallas_call(
        _proj_body,
        grid=(nby, r),
        in_specs=[
            pl.BlockSpec((by, d), lambda i, rr: (i, 0)),
            pl.BlockSpec((1, d, d), lambda i, rr: (rr, 0, 0)),
            pl.BlockSpec((1, 1, d), lambda i, rr: (rr, 0, 0)),
        ],
        out_specs=pl.BlockSpec((by, d), lambda i, rr: (rr * nby + i, 0)),
        out_shape=jax.ShapeDtypeStruct((r * n, d), jnp.float32),
    )(x, rel_W, rel_b.reshape(r, 1, d))


def _gidx_body(ei_ref, t_ref, g_ref):
    n = _gidx_body.n
    g_ref[...] = t_ref[...] * n + ei_ref[0]


def _gidx(ei5, typ4, n):
    _, nw, nsb, sbc, k = ei5.shape
    _gidx_body.n = n
    return pl.pallas_call(
        _gidx_body,
        grid=(nw // 8,),
        in_specs=[pl.BlockSpec((2, 8, nsb, sbc, k), lambda i: (0, i, 0, 0, 0)),
                  pl.BlockSpec((8, nsb, sbc, k), lambda i: (i, 0, 0, 0))],
        out_specs=pl.BlockSpec((8, nsb, sbc, k), lambda i: (i, 0, 0, 0)),
        out_shape=jax.ShapeDtypeStruct((nw, nsb, sbc, k), jnp.int32),
    )(ei5, typ4)


def _sc_aggregate(y, gidx4, ei5, n, d):
    sb = ei5.shape[3]
    zr = 1000
    nzt = n // zr
    mesh = plsc.VectorSubcoreMesh(core_axis_name="c", subcore_axis_name="s",
                                  num_cores=NC)

    @functools.partial(
        pl.kernel,
        out_type=[
            jax.ShapeDtypeStruct((NC, n, d), jnp.float32),
            jax.ShapeDtypeStruct((NC, 1, n), jnp.float32),
        ],
        mesh=mesh,
    scratch_types=[
            pltpu.VMEM((sb, K), jnp.int32),
            pltpu.VMEM((sb, K), jnp.int32),
            pltpu.VMEM((sb, K), jnp.int32),
            pltpu.VMEM((sb, K), jnp.int32),
            pltpu.VMEM((K, d), jnp.float32),
            pltpu.VMEM((K, d), jnp.float32),
            pltpu.VMEM((K, d), jnp.float32),
            pltpu.VMEM((K,), jnp.float32),
            pltpu.VMEM_SHARED((n, d), jnp.float32),
            pltpu.VMEM_SHARED((n,), jnp.float32),
            pltpu.SemaphoreType.DMA,
            pltpu.SemaphoreType.DMA,
            pltpu.SemaphoreType.DMA,
            pltpu.SemaphoreType.DMA,
            pltpu.SemaphoreType.DMA,
            pltpu.SemaphoreType.DMA,
            pltpu.SemaphoreType.DMA,
            pltpu.SemaphoreType.DMA,
        ],
    )
    def sc_kernel(y_hbm, gidx_hbm, ei_hbm, z2_hbm, z1_hbm,
                  acc_out, deg_out,
                  gidx0_v, gidx1_v, dst0_v, dst1_v,
                  rows0_v, rows1_v, rows2_v, ones_v,
                  acc_sh, deg_sh, semg0, semg1, semg2,
                  sems0, sems1, sems2, semd, semi):
        c = lax.axis_index("c")
        s = lax.axis_index("s")
        widx = c * NS + s

        arow = pl.multiple_of(s * zr, 8)

        @pl.when(s < nzt)
        def _():
            pltpu.sync_copy(z2_hbm, acc_sh.at[pl.ds(arow, zr)])

        @pl.when(s == 0)
        def _():
            pltpu.sync_copy(z1_hbm, deg_sh)

        for k in range(K // LANES):
            ones_v[pl.ds(k * LANES, LANES)] = jnp.ones((LANES,), jnp.float32)

        plsc.subcore_barrier()

        bufs = (rows0_v, rows1_v, rows2_v)
        semg = (semg0, semg1, semg2)
        sems = (sems0, sems1, sems2)
        gstage = (gidx0_v, gidx1_v)
        dstage = (dst0_v, dst1_v)

        def issue_stage(b, w):
            pltpu.async_copy(gidx_hbm.at[widx, b], gstage[w], semi)
            pltpu.async_copy(ei_hbm.at[1, widx, b], dstage[w], semi)

        def wait_stage(b, w):
            pltpu.make_async_copy(gidx_hbm.at[widx, b], gstage[w],
                                  semi).wait()
            pltpu.make_async_copy(ei_hbm.at[1, widx, b], dstage[w],
                                  semi).wait()

        issue_stage(0, 0)
        wait_stage(0, 0)

        for b in range(NSB):
            w = b % 2
            gidx_v = gstage[w]
            dst_v = dstage[w]
            if b + 1 < NSB:
                issue_stage(b + 1, 1 - w)

            def wait_g(j, p):
                pltpu.make_async_copy(y_hbm.at[gidx_v.at[j]], bufs[p],
                                      semg[p]).wait()

            def issue_s(j, p):
                pltpu.async_copy(bufs[p], acc_sh.at[dst_v.at[j]], sems[p],
                                 add=True)
                pltpu.async_copy(ones_v, deg_sh.at[dst_v.at[j]], semd,
                                 add=True)

            def wait_s(j, p):
                pltpu.make_async_copy(bufs[p], acc_sh.at[dst_v.at[j]],
                                      sems[p]).wait()

            pltpu.async_copy(y_hbm.at[gidx_v.at[0]], rows0_v, semg0)
            pltpu.async_copy(y_hbm.at[gidx_v.at[1]], rows1_v, semg1)
            pltpu.async_copy(y_hbm.at[gidx_v.at[2]], rows2_v, semg2)
            wait_g(0, 0)
            issue_s(0, 0)

            def tri(i, carry2):
                for t in range(3):
                    j = i * 3 + 1 + t
                    p = (1 + t) % 3
                    q = t % 3
                    wait_s(j - 1, q)

                    @pl.when(j + 2 < sb)
                    def _():
                        pltpu.async_copy(y_hbm.at[gidx_v.at[j + 2]],
                                         bufs[q], semg[q])
                    wait_g(j, p)
                    issue_s(j, p)
                return carry2
            lax.fori_loop(0, (sb - 1) // 3, tri, 0, unroll=False)
            wait_s(sb - 1, (sb - 1) % 3)

            def drain(j, carry2):
                pltpu.make_async_copy(ones_v, deg_sh.at[dst_v.at[j]],
                                      semd).wait()
                return carry2
            lax.fori_loop(0, sb, drain, 0, unroll=False)
            if b + 1 < NSB:
                wait_stage(b + 1, 1 - w)

        plsc.subcore_barrier()

        @pl.when(s < nzt)
        def _():
            pltpu.sync_copy(acc_sh.at[pl.ds(arow, zr)],
                            acc_out.at[c, pl.ds(arow, zr)])

        @pl.when(s == 0)
        def _():
            pltpu.sync_copy(deg_sh, deg_out.at[c, 0])

    z2 = jnp.zeros((zr, d), jnp.float32)
    z1 = jnp.zeros((n,), jnp.float32)
    return sc_kernel(y, gidx4, ei5, z2, z1)


def _ln(t, g, b, eps=1e-5):
    m = jnp.mean(t, axis=-1, keepdims=True)
    v = jnp.mean((t - m) ** 2, axis=-1, keepdims=True)
    return (t - m) / jnp.sqrt(v + eps) * g + b


def _tail_body(x_ref, a_ref, d_ref, sw_ref, sb_ref, g1_ref, bb1_ref,
               w1_ref, b1_ref, w2_ref, b2_ref, g2_ref, bb2_ref, o_ref):
    xb = x_ref[...]
    deg = jnp.maximum(sum(d_ref[i] for i in range(NC)), 1.0)
    agg = sum(a_ref[i] for i in range(NC)) / deg
    h = (lax.dot_general(xb, sw_ref[...], (((1,), (1,)), ((), ())),
                         preferred_element_type=jnp.float32)
         + sb_ref[...] + agg)
    x1 = _ln(xb + h, g1_ref[...], bb1_ref[...])
    u = (lax.dot_general(x1, w1_ref[...], (((1,), (1,)), ((), ())),
                         preferred_element_type=jnp.float32)
         + b1_ref[...])
    gl = 0.5 * u * (1.0 + lax.erf(u * 0.7071067811865476))
    ff = (lax.dot_general(gl, w2_ref[...], (((1,), (1,)), ((), ())),
                          preferred_element_type=jnp.float32)
          + b2_ref[...])
    o_ref[...] = _ln(x1 + ff, g2_ref[...], bb2_ref[...])


def _tail(x, accp, degp, self_W, self_b, ln1_g, ln1_b, W1, b1, W2, b2,
          ln2_g, ln2_b, bn=1000):
    n, d = x.shape
    h2 = W1.shape[0]
    nb = n // bn
    full = lambda *shape: pl.BlockSpec(shape, lambda i: (0,) * len(shape))
    return pl.pallas_call(
        _tail_body,
        grid=(nb,),
        in_specs=[
            pl.BlockSpec((bn, d), lambda i: (i, 0)),
            pl.BlockSpec((NC, bn, d), lambda i: (0, i, 0)),
            pl.BlockSpec((NC, bn, 1), lambda i: (0, i, 0)),
            full(d, d), full(1, d), full(1, d), full(1, d),
            full(h2, d), full(1, h2), full(d, h2), full(1, d),
            full(1, d), full(1, d),
        ],
        out_specs=pl.BlockSpec((bn, d), lambda i: (i, 0)),
        out_shape=jax.ShapeDtypeStruct((n, d), jnp.float32),
    )(x, accp, degp, self_W, self_b.reshape(1, d), ln1_g.reshape(1, d),
      ln1_b.reshape(1, d), W1, b1.reshape(1, h2), W2, b2.reshape(1, d),
      ln2_g.reshape(1, d), ln2_b.reshape(1, d))


def kernel(x, edge_index, edge_type, self_W, self_b, rel_W, rel_b,
           ln1_g, ln1_b, W1, b1, W2, b2, ln2_g, ln2_b):
    n, d = x.shape
    e = edge_type.shape[0]
    y = _project_all(x, rel_W, rel_b)
    nw = NC * NS
    ei5 = edge_index.reshape(2, nw, NSB, -1, K)
    typ4 = edge_type.reshape(nw, NSB, -1, K)
    gidx4 = _gidx(ei5, typ4, n)
    accp, degp = _sc_aggregate(y, gidx4, ei5, n, d)
    return _tail(x, accp, degp.reshape(NC, n, 1), self_W, self_b,
                 ln1_g, ln1_b, W1, b1, W2, b2, ln2_g, ln2_b)


# speedup vs baseline: 1.1269x; 1.0911x over previous
import functools

import jax
import jax.numpy as jnp
from jax import lax
from jax.experimental import pallas as pl
from jax.experimental.pallas import tpu as pltpu
from jax.experimental.pallas import tpu_sc as plsc

NC = 2
NSB = 5
NS = 16
K = 80
LANES = 16


def _proj_body(x_ref, w_ref, b_ref, y_ref):
    y_ref[...] = (
        lax.dot_general(x_ref[...], w_ref[0], (((1,), (1,)), ((), ())),
                        preferred_element_type=jnp.float32)
        + b_ref[0]
    )


def _project_all(x, rel_W, rel_b, by=2000):
    n, d = x.shape
    r = rel_W.shape[0]
    nby = n // by
    return pl.pallas_call(
        _proj_body,
        grid=(nby, r),
        in_specs=[
            pl.BlockSpec((by, d), lambda i, rr: (i, 0)),
            pl.BlockSpec((1, d, d), lambda i, rr: (rr, 0, 0)),
            pl.BlockSpec((1, 1, d), lambda i, rr: (rr, 0, 0)),
        ],
        out_specs=pl.BlockSpec((by, d), lambda i, rr: (rr * nby + i, 0)),
        out_shape=jax.ShapeDtypeStruct((r * n, d), jnp.float32),
    )(x, rel_W, rel_b.reshape(r, 1, d))


def _gidx_body(ei_ref, t_ref, g_ref):
    n = _gidx_body.n
    g_ref[...] = t_ref[...] * n + ei_ref[0]


def _gidx(ei5, typ4, n):
    _, nw, nsb, sbc, k = ei5.shape
    _gidx_body.n = n
    return pl.pallas_call(
        _gidx_body,
        grid=(nw // 8,),
        in_specs=[pl.BlockSpec((2, 8, nsb, sbc, k), lambda i: (0, i, 0, 0, 0)),
                  pl.BlockSpec((8, nsb, sbc, k), lambda i: (i, 0, 0, 0))],
        out_specs=pl.BlockSpec((8, nsb, sbc, k), lambda i: (i, 0, 0, 0)),
        out_shape=jax.ShapeDtypeStruct((nw, nsb, sbc, k), jnp.int32),
    )(ei5, typ4)


def _sc_aggregate(y, gidx4, ei5, n, d):
    sb = ei5.shape[3]
    zr = 1000
    nzt = n // zr
    mesh = plsc.VectorSubcoreMesh(core_axis_name="c", subcore_axis_name="s",
                                  num_cores=NC)

    @functools.partial(
        pl.kernel,
        out_type=[
            jax.ShapeDtypeStruct((NC, n, d), jnp.float32),
            jax.ShapeDtypeStruct((NC, 1, n), jnp.float32),
        ],
        mesh=mesh,
    scratch_types=[
            pltpu.VMEM((sb, K), jnp.int32),
            pltpu.VMEM((sb, K), jnp.int32),
            pltpu.VMEM((sb, K), jnp.int32),
            pltpu.VMEM((sb, K), jnp.int32),
            pltpu.VMEM((K, d), jnp.float32),
            pltpu.VMEM((K, d), jnp.float32),
            pltpu.VMEM((K, d), jnp.float32),
            pltpu.VMEM((K,), jnp.float32),
            pltpu.VMEM_SHARED((n, d), jnp.float32),
            pltpu.VMEM_SHARED((n,), jnp.float32),
            pltpu.SemaphoreType.DMA,
            pltpu.SemaphoreType.DMA,
            pltpu.SemaphoreType.DMA,
            pltpu.SemaphoreType.DMA,
            pltpu.SemaphoreType.DMA,
            pltpu.SemaphoreType.DMA,
            pltpu.SemaphoreType.DMA,
            pltpu.SemaphoreType.DMA,
        ],
    )
    def sc_kernel(y_hbm, gidx_hbm, ei_hbm, z2_hbm, z1_hbm,
                  acc_out, deg_out,
                  gidx0_v, gidx1_v, dst0_v, dst1_v,
                  rows0_v, rows1_v, rows2_v, ones_v,
                  acc_sh, deg_sh, semg0, semg1, semg2,
                  sems0, sems1, sems2, semd, semi):
        c = lax.axis_index("c")
        s = lax.axis_index("s")
        widx = c * NS + s

        arow = pl.multiple_of(s * zr, 8)

        @pl.when(s < nzt)
        def _():
            pltpu.sync_copy(z2_hbm, acc_sh.at[pl.ds(arow, zr)])

        @pl.when(s == 0)
        def _():
            pltpu.sync_copy(z1_hbm, deg_sh)

        for k in range(K // LANES):
            ones_v[pl.ds(k * LANES, LANES)] = jnp.ones((LANES,), jnp.float32)

        plsc.subcore_barrier()

        bufs = (rows0_v, rows1_v, rows2_v)
        semg = (semg0, semg1, semg2)
        sems = (sems0, sems1, sems2)
        gstage = (gidx0_v, gidx1_v)
        dstage = (dst0_v, dst1_v)

        def issue_stage(b, w):
            pltpu.async_copy(gidx_hbm.at[widx, b], gstage[w], semi)
            pltpu.async_copy(ei_hbm.at[1, widx, b], dstage[w], semi)

        def wait_stage(b, w):
            pltpu.make_async_copy(gidx_hbm.at[widx, b], gstage[w],
                                  semi).wait()
            pltpu.make_async_copy(ei_hbm.at[1, widx, b], dstage[w],
                                  semi).wait()

        issue_stage(0, 0)
        wait_stage(0, 0)

        for b in range(NSB):
            w = b % 2
            gidx_v = gstage[w]
            dst_v = dstage[w]
            if b + 1 < NSB:
                issue_stage(b + 1, 1 - w)

            def wait_g(j, p):
                pltpu.make_async_copy(y_hbm.at[gidx_v.at[j]], bufs[p],
                                      semg[p]).wait()

            def issue_s(j, p):
                pltpu.async_copy(bufs[p], acc_sh.at[dst_v.at[j]], sems[p],
                                 add=True)
                pltpu.async_copy(ones_v, deg_sh.at[dst_v.at[j]], semd,
                                 add=True)

            def wait_s(j, p):
                pltpu.make_async_copy(bufs[p], acc_sh.at[dst_v.at[j]],
                                      sems[p]).wait()

            pltpu.async_copy(y_hbm.at[gidx_v.at[0]], rows0_v, semg0)
            pltpu.async_copy(y_hbm.at[gidx_v.at[1]], rows1_v, semg1)
            pltpu.async_copy(y_hbm.at[gidx_v.at[2]], rows2_v, semg2)
            wait_g(0, 0)
            issue_s(0, 0)

            def tri(i, carry2):
                for t in range(3):
                    j = i * 3 + 1 + t
                    p = (1 + t) % 3
                    q = t % 3
                    wait_s(j - 1, q)

                    @pl.when(j + 2 < sb)
                    def _():
                        pltpu.async_copy(y_hbm.at[gidx_v.at[j + 2]],
                                         bufs[q], semg[q])
                    wait_g(j, p)
                    issue_s(j, p)
                return carry2
            lax.fori_loop(0, (sb - 1) // 3, tri, 0, unroll=False)
            wait_s(sb - 1, (sb - 1) % 3)

            def drain(j, carry2):
                pltpu.make_async_copy(ones_v, deg_sh.at[dst_v.at[j]],
                                      semd).wait()
                return carry2
            lax.fori_loop(0, sb, drain, 0, unroll=False)
            if b + 1 < NSB:
                wait_stage(b + 1, 1 - w)

        plsc.subcore_barrier()

        @pl.when(s < nzt)
        def _():
            pltpu.sync_copy(acc_sh.at[pl.ds(arow, zr)],
                            acc_out.at[c, pl.ds(arow, zr)])

        @pl.when(s == 0)
        def _():
            pltpu.sync_copy(deg_sh, deg_out.at[c, 0])

    z2 = jnp.zeros((zr, d), jnp.float32)
    z1 = jnp.zeros((n,), jnp.float32)
    return sc_kernel(y, gidx4, ei5, z2, z1)


def _ln(t, g, b, eps=1e-5):
    m = jnp.mean(t, axis=-1, keepdims=True)
    v = jnp.mean((t - m) ** 2, axis=-1, keepdims=True)
    return (t - m) / jnp.sqrt(v + eps) * g + b


def _tail_body(x_ref, a_ref, d_ref, sw_ref, sb_ref, g1_ref, bb1_ref,
               w1_ref, b1_ref, w2_ref, b2_ref, g2_ref, bb2_ref, o_ref):
    xb = x_ref[...]
    deg = jnp.maximum(sum(d_ref[i] for i in range(NC)), 1.0)
    agg = sum(a_ref[i] for i in range(NC)) / deg
    h = (lax.dot_general(xb, sw_ref[...], (((1,), (1,)), ((), ())),
                         preferred_element_type=jnp.float32)
         + sb_ref[...] + agg)
    x1 = _ln(xb + h, g1_ref[...], bb1_ref[...])
    u = (lax.dot_general(x1, w1_ref[...], (((1,), (1,)), ((), ())),
                         preferred_element_type=jnp.float32)
         + b1_ref[...])
    gl = 0.5 * u * (1.0 + lax.erf(u * 0.7071067811865476))
    ff = (lax.dot_general(gl, w2_ref[...], (((1,), (1,)), ((), ())),
                          preferred_element_type=jnp.float32)
          + b2_ref[...])
    o_ref[...] = _ln(x1 + ff, g2_ref[...], bb2_ref[...])


def _tail(x, accp, degp, self_W, self_b, ln1_g, ln1_b, W1, b1, W2, b2,
          ln2_g, ln2_b, bn=2000):
    n, d = x.shape
    h2 = W1.shape[0]
    nb = n // bn
    full = lambda *shape: pl.BlockSpec(shape, lambda i: (0,) * len(shape))
    return pl.pallas_call(
        _tail_body,
        grid=(nb,),
        in_specs=[
            pl.BlockSpec((bn, d), lambda i: (i, 0)),
            pl.BlockSpec((NC, bn, d), lambda i: (0, i, 0)),
            pl.BlockSpec((NC, bn, 1), lambda i: (0, i, 0)),
            full(d, d), full(1, d), full(1, d), full(1, d),
            full(h2, d), full(1, h2), full(d, h2), full(1, d),
            full(1, d), full(1, d),
        ],
        out_specs=pl.BlockSpec((bn, d), lambda i: (i, 0)),
        out_shape=jax.ShapeDtypeStruct((n, d), jnp.float32),
    )(x, accp, degp, self_W, self_b.reshape(1, d), ln1_g.reshape(1, d),
      ln1_b.reshape(1, d), W1, b1.reshape(1, h2), W2, b2.reshape(1, d),
      ln2_g.reshape(1, d), ln2_b.reshape(1, d))


def kernel(x, edge_index, edge_type, self_W, self_b, rel_W, rel_b,
           ln1_g, ln1_b, W1, b1, W2, b2, ln2_g, ln2_b):
    n, d = x.shape
    e = edge_type.shape[0]
    y = _project_all(x, rel_W, rel_b)
    nw = NC * NS
    ei5 = edge_index.reshape(2, nw, NSB, -1, K)
    typ4 = edge_type.reshape(nw, NSB, -1, K)
    gidx4 = _gidx(ei5, typ4, n)
    accp, degp = _sc_aggregate(y, gidx4, ei5, n, d)
    return _tail(x, accp, degp.reshape(NC, n, 1), self_W, self_b,
                 ln1_g, ln1_b, W1, b1, W2, b2, ln2_g, ln2_b)


# speedup vs baseline: 1.1480x; 1.0187x over previous
import functools

import jax
import jax.numpy as jnp
from jax import lax
from jax.experimental import pallas as pl
from jax.experimental.pallas import tpu as pltpu
from jax.experimental.pallas import tpu_sc as plsc

NC = 2
NSB = 5
NS = 16
K = 80
LANES = 16


def _proj_body(x_ref, w_ref, b_ref, y_ref):
    y_ref[...] = (
        lax.dot_general(x_ref[...], w_ref[0], (((1,), (1,)), ((), ())),
                        preferred_element_type=jnp.float32)
        + b_ref[0]
    )


def _project_all(x, rel_W, rel_b, by=5000):
    n, d = x.shape
    r = rel_W.shape[0]
    nby = n // by
    return pl.pallas_call(
        _proj_body,
        grid=(nby, r),
        in_specs=[
            pl.BlockSpec((by, d), lambda i, rr: (i, 0)),
            pl.BlockSpec((1, d, d), lambda i, rr: (rr, 0, 0)),
            pl.BlockSpec((1, 1, d), lambda i, rr: (rr, 0, 0)),
        ],
        out_specs=pl.BlockSpec((by, d), lambda i, rr: (rr * nby + i, 0)),
        out_shape=jax.ShapeDtypeStruct((r * n, d), jnp.float32),
    )(x, rel_W, rel_b.reshape(r, 1, d))


def _gidx_body(ei_ref, t_ref, g_ref):
    n = _gidx_body.n
    g_ref[...] = t_ref[...] * n + ei_ref[0]


def _gidx(ei5, typ4, n):
    _, nw, nsb, sbc, k = ei5.shape
    _gidx_body.n = n
    return pl.pallas_call(
        _gidx_body,
        grid=(nw // 8,),
        in_specs=[pl.BlockSpec((2, 8, nsb, sbc, k), lambda i: (0, i, 0, 0, 0)),
                  pl.BlockSpec((8, nsb, sbc, k), lambda i: (i, 0, 0, 0))],
        out_specs=pl.BlockSpec((8, nsb, sbc, k), lambda i: (i, 0, 0, 0)),
        out_shape=jax.ShapeDtypeStruct((nw, nsb, sbc, k), jnp.int32),
    )(ei5, typ4)


def _sc_aggregate(y, gidx4, ei5, n, d):
    sb = ei5.shape[3]
    zr = 1000
    nzt = n // zr
    mesh = plsc.VectorSubcoreMesh(core_axis_name="c", subcore_axis_name="s",
                                  num_cores=NC)

    @functools.partial(
        pl.kernel,
        out_type=[
            jax.ShapeDtypeStruct((NC, n, d), jnp.float32),
            jax.ShapeDtypeStruct((NC, 1, n), jnp.float32),
        ],
        mesh=mesh,
    scratch_types=[
            pltpu.VMEM((sb, K), jnp.int32),
            pltpu.VMEM((sb, K), jnp.int32),
            pltpu.VMEM((sb, K), jnp.int32),
            pltpu.VMEM((sb, K), jnp.int32),
            pltpu.VMEM((K, d), jnp.float32),
            pltpu.VMEM((K, d), jnp.float32),
            pltpu.VMEM((K, d), jnp.float32),
            pltpu.VMEM((K,), jnp.float32),
            pltpu.VMEM_SHARED((n, d), jnp.float32),
            pltpu.VMEM_SHARED((n,), jnp.float32),
            pltpu.SemaphoreType.DMA,
            pltpu.SemaphoreType.DMA,
            pltpu.SemaphoreType.DMA,
            pltpu.SemaphoreType.DMA,
            pltpu.SemaphoreType.DMA,
            pltpu.SemaphoreType.DMA,
            pltpu.SemaphoreType.DMA,
            pltpu.SemaphoreType.DMA,
        ],
    )
    def sc_kernel(y_hbm, gidx_hbm, ei_hbm, z2_hbm, z1_hbm,
                  acc_out, deg_out,
                  gidx0_v, gidx1_v, dst0_v, dst1_v,
                  rows0_v, rows1_v, rows2_v, ones_v,
                  acc_sh, deg_sh, semg0, semg1, semg2,
                  sems0, sems1, sems2, semd, semi):
        c = lax.axis_index("c")
        s = lax.axis_index("s")
        widx = c * NS + s

        arow = pl.multiple_of(s * zr, 8)

        @pl.when(s < nzt)
        def _():
            pltpu.sync_copy(z2_hbm, acc_sh.at[pl.ds(arow, zr)])

        @pl.when(s == 0)
        def _():
            pltpu.sync_copy(z1_hbm, deg_sh)

        for k in range(K // LANES):
            ones_v[pl.ds(k * LANES, LANES)] = jnp.ones((LANES,), jnp.float32)

        plsc.subcore_barrier()

        bufs = (rows0_v, rows1_v, rows2_v)
        semg = (semg0, semg1, semg2)
        sems = (sems0, sems1, sems2)
        gstage = (gidx0_v, gidx1_v)
        dstage = (dst0_v, dst1_v)

        def issue_stage(b, w):
            pltpu.async_copy(gidx_hbm.at[widx, b], gstage[w], semi)
            pltpu.async_copy(ei_hbm.at[1, widx, b], dstage[w], semi)

        def wait_stage(b, w):
            pltpu.make_async_copy(gidx_hbm.at[widx, b], gstage[w],
                                  semi).wait()
            pltpu.make_async_copy(ei_hbm.at[1, widx, b], dstage[w],
                                  semi).wait()

        issue_stage(0, 0)
        wait_stage(0, 0)

        for b in range(NSB):
            w = b % 2
            gidx_v = gstage[w]
            dst_v = dstage[w]
            if b + 1 < NSB:
                issue_stage(b + 1, 1 - w)

            def wait_g(j, p):
                pltpu.make_async_copy(y_hbm.at[gidx_v.at[j]], bufs[p],
                                      semg[p]).wait()

            def issue_s(j, p):
                pltpu.async_copy(bufs[p], acc_sh.at[dst_v.at[j]], sems[p],
                                 add=True)
                pltpu.async_copy(ones_v, deg_sh.at[dst_v.at[j]], semd,
                                 add=True)

            def wait_s(j, p):
                pltpu.make_async_copy(bufs[p], acc_sh.at[dst_v.at[j]],
                                      sems[p]).wait()

            pltpu.async_copy(y_hbm.at[gidx_v.at[0]], rows0_v, semg0)
            pltpu.async_copy(y_hbm.at[gidx_v.at[1]], rows1_v, semg1)
            pltpu.async_copy(y_hbm.at[gidx_v.at[2]], rows2_v, semg2)
            wait_g(0, 0)
            issue_s(0, 0)

            def tri(i, carry2):
                for t in range(3):
                    j = i * 3 + 1 + t
                    p = (1 + t) % 3
                    q = t % 3
                    wait_s(j - 1, q)

                    @pl.when(j + 2 < sb)
                    def _():
                        pltpu.async_copy(y_hbm.at[gidx_v.at[j + 2]],
                                         bufs[q], semg[q])
                    wait_g(j, p)
                    issue_s(j, p)
                return carry2
            lax.fori_loop(0, (sb - 1) // 3, tri, 0, unroll=False)
            wait_s(sb - 1, (sb - 1) % 3)

            def drain(j, carry2):
                pltpu.make_async_copy(ones_v, deg_sh.at[dst_v.at[j]],
                                      semd).wait()
                return carry2
            lax.fori_loop(0, sb, drain, 0, unroll=False)
            if b + 1 < NSB:
                wait_stage(b + 1, 1 - w)

        plsc.subcore_barrier()

        @pl.when(s < nzt)
        def _():
            pltpu.sync_copy(acc_sh.at[pl.ds(arow, zr)],
                            acc_out.at[c, pl.ds(arow, zr)])

        @pl.when(s == 0)
        def _():
            pltpu.sync_copy(deg_sh, deg_out.at[c, 0])

    z2 = jnp.zeros((zr, d), jnp.float32)
    z1 = jnp.zeros((n,), jnp.float32)
    return sc_kernel(y, gidx4, ei5, z2, z1)


def _ln(t, g, b, eps=1e-5):
    m = jnp.mean(t, axis=-1, keepdims=True)
    v = jnp.mean((t - m) ** 2, axis=-1, keepdims=True)
    return (t - m) / jnp.sqrt(v + eps) * g + b


def _tail_body(x_ref, a_ref, d_ref, sw_ref, sb_ref, g1_ref, bb1_ref,
               w1_ref, b1_ref, w2_ref, b2_ref, g2_ref, bb2_ref, o_ref):
    xb = x_ref[...]
    deg = jnp.maximum(sum(d_ref[i] for i in range(NC)), 1.0)
    agg = sum(a_ref[i] for i in range(NC)) / deg
    h = (lax.dot_general(xb, sw_ref[...], (((1,), (1,)), ((), ())),
                         preferred_element_type=jnp.float32)
         + sb_ref[...] + agg)
    x1 = _ln(xb + h, g1_ref[...], bb1_ref[...])
    u = (lax.dot_general(x1, w1_ref[...], (((1,), (1,)), ((), ())),
                         preferred_element_type=jnp.float32)
         + b1_ref[...])
    gl = 0.5 * u * (1.0 + lax.erf(u * 0.7071067811865476))
    ff = (lax.dot_general(gl, w2_ref[...], (((1,), (1,)), ((), ())),
                          preferred_element_type=jnp.float32)
          + b2_ref[...])
    o_ref[...] = _ln(x1 + ff, g2_ref[...], bb2_ref[...])


def _tail(x, accp, degp, self_W, self_b, ln1_g, ln1_b, W1, b1, W2, b2,
          ln2_g, ln2_b, bn=5000):
    n, d = x.shape
    h2 = W1.shape[0]
    nb = n // bn
    full = lambda *shape: pl.BlockSpec(shape, lambda i: (0,) * len(shape))
    return pl.pallas_call(
        _tail_body,
        grid=(nb,),
        in_specs=[
            pl.BlockSpec((bn, d), lambda i: (i, 0)),
            pl.BlockSpec((NC, bn, d), lambda i: (0, i, 0)),
            pl.BlockSpec((NC, bn, 1), lambda i: (0, i, 0)),
            full(d, d), full(1, d), full(1, d), full(1, d),
            full(h2, d), full(1, h2), full(d, h2), full(1, d),
            full(1, d), full(1, d),
        ],
        out_specs=pl.BlockSpec((bn, d), lambda i: (i, 0)),
        out_shape=jax.ShapeDtypeStruct((n, d), jnp.float32),
    )(x, accp, degp, self_W, self_b.reshape(1, d), ln1_g.reshape(1, d),
      ln1_b.reshape(1, d), W1, b1.reshape(1, h2), W2, b2.reshape(1, d),
      ln2_g.reshape(1, d), ln2_b.reshape(1, d))


def kernel(x, edge_index, edge_type, self_W, self_b, rel_W, rel_b,
           ln1_g, ln1_b, W1, b1, W2, b2, ln2_g, ln2_b):
    n, d = x.shape
    e = edge_type.shape[0]
    y = _project_all(x, rel_W, rel_b)
    nw = NC * NS
    ei5 = edge_index.reshape(2, nw, NSB, -1, K)
    typ4 = edge_type.reshape(nw, NSB, -1, K)
    gidx4 = _gidx(ei5, typ4, n)
    accp, degp = _sc_aggregate(y, gidx4, ei5, n, d)
    return _tail(x, accp, degp.reshape(NC, n, 1), self_W, self_b,
                 ln1_g, ln1_b, W1, b1, W2, b2, ln2_g, ln2_b)
